# baseline (speedup 1.0000x reference)
import functools

import jax
import jax.numpy as jnp
from jax import lax
from jax.experimental import pallas as pl
from jax.experimental.pallas import tpu as pltpu
from jax.experimental.pallas import tpu_sc as plsc

_CH = 128
_UNROLL = 8
_SC_PARAMS = dict(
    compiler_params=None,
)


def _sc_info():
    try:
        info = plsc.get_sparse_core_info()
        return info.num_cores, info.num_subcores
    except Exception:
        return 2, 16


def _sc_cp():
    return pltpu.CompilerParams(
        needs_layout_passes=False, use_tc_tiling_on_sc=False)


def _make_gather(L, FP, B, NC, NS):
    NW = NC * NS
    bpw = B // NW
    nch = bpw // _CH
    mesh = plsc.VectorSubcoreMesh(core_axis_name="c", subcore_axis_name="s")

    @functools.partial(
        pl.kernel,
        out_type=jax.ShapeDtypeStruct((B, FP), jnp.float32),
        mesh=mesh,
        scratch_types=[
            pltpu.VMEM((nch, _CH), jnp.int32),
            pltpu.VMEM((2, _CH, FP), jnp.float32),
            pltpu.VMEM((nch, _CH), jnp.int32),
            pltpu.SemaphoreType.DMA,
            pltpu.SemaphoreType.DMA,
        ],
        compiler_params=_sc_cp(),
    )
    def k(fb, lb, ind2, fo_out, idx4, rows2, lbl4, semr, seml):
        cid = lax.axis_index("c")
        sid = lax.axis_index("s")
        w = sid * NC + cid
        base = w * bpw
        lanes = lax.iota(jnp.int32, 16)
        pltpu.sync_copy(ind2.at[pl.ds(w * nch, nch)], idx4)
        lg = [pltpu.async_copy(lb.at[idx4.at[c]], lbl4.at[c], seml)
              for c in range(nch)]
        gd = [None, None]

        def fire(c):
            gd[c % 2] = pltpu.async_copy(fb.at[idx4.at[c]], rows2.at[c % 2],
                                         semr)

        def put(c):
            s = c % 2
            gd[s].wait()
            lg[c].wait()
            for j in range(_CH // 16):
                lv = plsc.bitcast(lbl4[c, pl.ds(j * 16, 16)], jnp.float32)
                plsc.store_scatter(
                    rows2, [jnp.full((16,), s, jnp.int32),
                            j * 16 + lanes,
                            jnp.full((16,), 64, jnp.int32)], lv)
            pltpu.sync_copy(rows2.at[s],
                            fo_out.at[pl.ds(base + c * _CH, _CH)])

        fire(0)
        for c in range(nch):
            if c + 1 < nch:
                fire(c + 1)
            put(c)

    return k


def _make_stamp(L, B, NC, NS):
    NW = NC * NS
    HB = B // 2
    lcw = ((L // NW) + 7) // 8 * 8
    mesh = plsc.VectorSubcoreMesh(core_axis_name="c", subcore_axis_name="s")

    @functools.partial(
        pl.kernel,
        out_type=(
            jax.ShapeDtypeStruct((L,), jnp.int32),
            jax.ShapeDtypeStruct((L,), jnp.int32),
        ),
        mesh=mesh,
        scratch_types=[
            pltpu.VMEM((HB // _CH, _CH), jnp.int32),
            pltpu.VMEM((L,), jnp.int32),
        ],
        compiler_params=_sc_cp(),
    )
    def k(lb, ind2, st_out, lbc_out, indh, stampv):
        cid = lax.axis_index("c")
        sid = lax.axis_index("s")
        w = sid * NC + cid
        lanes = lax.iota(jnp.int32, 16)
        lco = w * lcw
        ltail = L - (NW - 1) * lcw

        @pl.when(w < NW - 1)
        def _copy_body():
            pltpu.sync_copy(lb.at[pl.ds(lco, lcw)],
                            lbc_out.at[pl.ds(lco, lcw)])

        @pl.when(w == NW - 1)
        def _copy_tail():
            pltpu.sync_copy(lb.at[pl.ds(lco, ltail)],
                            lbc_out.at[pl.ds(lco, ltail)])

        @pl.when(w == 0)
        def _():
            nhr = HB // _CH
            for h in range(2):
                pltpu.sync_copy(ind2.at[pl.ds(h * nhr, nhr)], indh)

                def body(i, carry):
                    gbase = h * HB + i * _CH
                    idxs, bs = [], []
                    for u in range(_UNROLL):
                        idx16 = indh[i, pl.ds(u * 16, 16)]
                        b16 = gbase + u * 16 + lanes
                        plsc.store_scatter(stampv, [idx16], b16)
                        idxs.append(idx16)
                        bs.append(b16)
                    ok = None
                    for u in range(_UNROLL):
                        g = plsc.load_gather(stampv, [idxs[u]])
                        e = g == bs[u]
                        ok = e if ok is None else jnp.logical_and(ok, e)
                    dup = jnp.logical_not(jnp.all(ok))

                    @pl.when(dup)
                    def _fix():
                        for u in range(_UNROLL):
                            for kk in range(16):
                                plsc.store_scatter(stampv, [idxs[u]], bs[u],
                                                   mask=lanes == kk)

                    return carry

                lax.fori_loop(0, nhr, body, 0)
            pltpu.sync_copy(stampv, st_out)

    return k


def _make_scatter(L, FP, B, NC, NS):
    NW = NC * NS
    bpw = B // NW
    nch = bpw // _CH
    mesh = plsc.VectorSubcoreMesh(core_axis_name="c", subcore_axis_name="s")

    @functools.partial(
        pl.kernel,
        out_type=(),
        mesh=mesh,
        scratch_types=[
            pltpu.VMEM((nch, _CH), jnp.int32),
            pltpu.VMEM((nch, _CH), jnp.int32),
            pltpu.VMEM((nch, _CH, FP), jnp.float32),
            pltpu.VMEM((nch, _CH), jnp.int32),
            pltpu.SemaphoreType.DMA,
            pltpu.SemaphoreType.DMA,
        ],
        compiler_params=_sc_cp(),
    )
    def k(fb_ref, lb_ref, ind2, stamp, fnew, idx4, gc4, rows4, lbl4,
          semr, seml):
        cid = lax.axis_index("c")
        sid = lax.axis_index("s")
        w = sid * NC + cid
        lanes = lax.iota(jnp.int32, 16)
        pltpu.sync_copy(ind2.at[pl.ds(w * nch, nch)], idx4)
        sg = [pltpu.async_copy(stamp.at[idx4.at[c]], gc4.at[c], seml)
              for c in range(nch)]
        for d in sg:
            d.wait()
        rg = [pltpu.async_copy(fnew.at[gc4.at[c]], rows4.at[c], semr)
              for c in range(nch)]
        for c in range(nch):
            rg[c].wait()
            for j in range(_CH // 16):
                lv = plsc.load_gather(
                    rows4, [jnp.full((16,), c, jnp.int32),
                            j * 16 + lanes,
                            jnp.full((16,), 64, jnp.int32)])
                lbl4[c, pl.ds(j * 16, 16)] = plsc.bitcast(lv, jnp.int32)
        rs = [pltpu.async_copy(rows4.at[c], fb_ref.at[idx4.at[c]], semr)
              for c in range(nch)]
        ls = [pltpu.async_copy(lbl4.at[c], lb_ref.at[idx4.at[c]], seml)
              for c in range(nch)]
        for d in rs:
            d.wait()
        for d in ls:
            d.wait()

    return k


def _make_dense(C, F, FP, B, BB):
    G = B // BB

    def body(f_ref, fo_ref, c_ref, fn_ref, cs_ref):
        pid = pl.program_id(0)
        f = f_ref[...]
        foe = fo_ref[...]
        fo = foe[:, :F]
        ol = lax.bitcast_convert_type(foe[:, F:F + 1], jnp.int32)
        cen = c_ref[...]
        fn = f / (jnp.sqrt(jnp.sum(f * f, axis=1, keepdims=True)) + 1e-10)
        fnew = 0.5 * fo + 0.5 * fn
        fnew = fnew / (jnp.sqrt(jnp.sum(fnew * fnew, axis=1, keepdims=True))
                       + 1e-10)
        sims = lax.dot_general(fnew, cen, (((1,), (1,)), ((), ())),
                               preferred_element_type=jnp.float32)
        m = jnp.max(sims, axis=1, keepdims=True)
        cio = lax.broadcasted_iota(jnp.int32, sims.shape, 1)
        pick = jnp.where(sims == m, cio, jnp.int32(2 ** 30))
        lbl = jnp.min(pick, axis=1, keepdims=True)
        pad = jnp.zeros((BB, FP - F - 1), jnp.float32)
        fn_ref[...] = jnp.concatenate(
            [fnew, lax.bitcast_convert_type(lbl, jnp.float32), pad], axis=1)
        neq = (lbl != ol).astype(jnp.float32)
        s = jnp.sum(neq, axis=0, keepdims=True)

        @pl.when(pid == 0)
        def _():
            cs_ref[...] = jnp.zeros((1, 1), jnp.float32)

        cs_ref[...] += s * (1.0 / B)

    return pl.pallas_call(
        body,
        grid=(G,),
        in_specs=[
            pl.BlockSpec((BB, F), lambda i: (i, 0)),
            pl.BlockSpec((BB, FP), lambda i: (i, 0)),
            pl.BlockSpec((C, F), lambda i: (0, 0)),
        ],
        out_specs=[
            pl.BlockSpec((BB, FP), lambda i: (i, 0)),
            pl.BlockSpec((1, 1), lambda i: (0, 0)),
        ],
        out_shape=[
            jax.ShapeDtypeStruct((B, FP), jnp.float32),
            jax.ShapeDtypeStruct((1, 1), jnp.float32),
        ],
    )


def kernel(feature_bank, centroids, feature, label_bank, ind):
    L, F = feature_bank.shape
    C = centroids.shape[0]
    B = ind.shape[0]
    FP = 128
    NC, NS = _sc_info()
    BB = 1024

    sel = jnp.concatenate(
        [jnp.eye(F, dtype=jnp.float32),
         jnp.zeros((F, FP - F), jnp.float32)], axis=1)
    fb128 = jax.lax.dot_general(
        feature_bank, sel, (((1,), (0,)), ((), ())),
        precision=jax.lax.Precision.HIGHEST,
        preferred_element_type=jnp.float32)

    ind2 = ind.astype(jnp.int32).reshape(B // _CH, _CH)
    stamp, lbc = _make_stamp(L, B, NC, NS)(label_bank, ind2)
    fo128 = _make_gather(L, FP, B, NC, NS)(fb128, label_bank, ind2)
    fnew128, cs = _make_dense(C, F, FP, B, BB)(feature, fo128, centroids)

    fb_ref = jax.new_ref(fb128)
    lb_ref = jax.new_ref(lbc)
    _make_scatter(L, FP, B, NC, NS)(fb_ref, lb_ref, ind2, stamp, fnew128)
    return fb_ref[...][:, :F], lb_ref[...], cs[0, 0]


# speedup vs baseline: 1.1767x; 1.1767x over previous
import functools

import jax
import jax.numpy as jnp
from jax import lax
from jax.experimental import pallas as pl
from jax.experimental.pallas import tpu as pltpu
from jax.experimental.pallas import tpu_sc as plsc

_CH = 128
_UNROLL = 8
_SC_PARAMS = dict(
    compiler_params=None,
)


def _sc_info():
    try:
        info = plsc.get_sparse_core_info()
        return info.num_cores, info.num_subcores
    except Exception:
        return 2, 16


def _sc_cp():
    return pltpu.CompilerParams(
        needs_layout_passes=False, use_tc_tiling_on_sc=False)


def _make_gather(L, FP, B, NC, NS):
    NW = NC * NS
    bpw = B // NW
    nch = bpw // _CH
    mesh = plsc.VectorSubcoreMesh(core_axis_name="c", subcore_axis_name="s")

    @functools.partial(
        pl.kernel,
        out_type=jax.ShapeDtypeStruct((B, FP), jnp.float32),
        mesh=mesh,
        scratch_types=[
            pltpu.VMEM((nch, _CH), jnp.int32),
            pltpu.VMEM((2, _CH, FP), jnp.float32),
            pltpu.VMEM((nch, _CH), jnp.int32),
            pltpu.SemaphoreType.DMA,
            pltpu.SemaphoreType.DMA,
        ],
        compiler_params=_sc_cp(),
    )
    def k(fb, lb, ind2, fo_out, idx4, rows2, lbl4, semr, seml):
        cid = lax.axis_index("c")
        sid = lax.axis_index("s")
        w = sid * NC + cid
        base = w * bpw
        lanes = lax.iota(jnp.int32, 16)
        pltpu.sync_copy(ind2.at[pl.ds(w * nch, nch)], idx4)
        lg = [pltpu.async_copy(lb.at[idx4.at[c]], lbl4.at[c], seml)
              for c in range(nch)]
        gd = [None, None]

        def fire(c):
            gd[c % 2] = pltpu.async_copy(fb.at[idx4.at[c]], rows2.at[c % 2],
                                         semr)

        def put(c):
            s = c % 2
            gd[s].wait()
            lg[c].wait()
            for j in range(_CH // 16):
                lv = plsc.bitcast(lbl4[c, pl.ds(j * 16, 16)], jnp.float32)
                plsc.store_scatter(
                    rows2, [jnp.full((16,), s, jnp.int32),
                            j * 16 + lanes,
                            jnp.full((16,), 64, jnp.int32)], lv)
            pltpu.sync_copy(rows2.at[s],
                            fo_out.at[pl.ds(base + c * _CH, _CH)])

        fire(0)
        for c in range(nch):
            if c + 1 < nch:
                fire(c + 1)
            put(c)

    return k


def _make_stamp(L, B, NC, NS):
    NW = NC * NS
    HB = B // 2
    lcw = ((L // NW) + 7) // 8 * 8
    mesh = plsc.VectorSubcoreMesh(core_axis_name="c", subcore_axis_name="s")

    @functools.partial(
        pl.kernel,
        out_type=(
            jax.ShapeDtypeStruct((L,), jnp.int32),
            jax.ShapeDtypeStruct((L,), jnp.int32),
        ),
        mesh=mesh,
        scratch_types=[
            pltpu.VMEM((HB // _CH, _CH), jnp.int32),
            pltpu.VMEM((L,), jnp.int32),
        ],
        compiler_params=_sc_cp(),
    )
    def k(lb, ind2, st_out, lbc_out, indh, stampv):
        cid = lax.axis_index("c")
        sid = lax.axis_index("s")
        w = sid * NC + cid
        lanes = lax.iota(jnp.int32, 16)
        lco = w * lcw
        ltail = L - (NW - 1) * lcw

        @pl.when(w < NW - 1)
        def _copy_body():
            pltpu.sync_copy(lb.at[pl.ds(lco, lcw)],
                            lbc_out.at[pl.ds(lco, lcw)])

        @pl.when(w == NW - 1)
        def _copy_tail():
            pltpu.sync_copy(lb.at[pl.ds(lco, ltail)],
                            lbc_out.at[pl.ds(lco, ltail)])

        @pl.when(w == 0)
        def _():
            nhr = HB // _CH
            for h in range(2):
                pltpu.sync_copy(ind2.at[pl.ds(h * nhr, nhr)], indh)

                def body(i, carry):
                    gbase = h * HB + i * _CH
                    idxs, bs = [], []
                    for u in range(_UNROLL):
                        idx16 = indh[i, pl.ds(u * 16, 16)]
                        b16 = gbase + u * 16 + lanes
                        plsc.store_scatter(stampv, [idx16], b16)
                        idxs.append(idx16)
                        bs.append(b16)
                    ok = None
                    for u in range(_UNROLL):
                        g = plsc.load_gather(stampv, [idxs[u]])
                        e = g == bs[u]
                        ok = e if ok is None else jnp.logical_and(ok, e)
                    dup = jnp.logical_not(jnp.all(ok))

                    @pl.when(dup)
                    def _fix():
                        for u in range(_UNROLL):
                            for kk in range(16):
                                plsc.store_scatter(stampv, [idxs[u]], bs[u],
                                                   mask=lanes == kk)

                    return carry

                lax.fori_loop(0, nhr, body, 0)
            pltpu.sync_copy(stampv, st_out)

    return k


def _make_scatter(L, FP, B, NC, NS):
    NW = NC * NS
    bpw = B // NW
    nch = bpw // _CH
    mesh = plsc.VectorSubcoreMesh(core_axis_name="c", subcore_axis_name="s")

    @functools.partial(
        pl.kernel,
        out_type=(),
        mesh=mesh,
        scratch_types=[
            pltpu.VMEM((nch, _CH), jnp.int32),
            pltpu.VMEM((nch, _CH), jnp.int32),
            pltpu.VMEM((nch, _CH, FP), jnp.float32),
            pltpu.VMEM((nch, _CH), jnp.int32),
            pltpu.SemaphoreType.DMA,
            pltpu.SemaphoreType.DMA,
        ],
        compiler_params=_sc_cp(),
    )
    def k(fb_ref, lb_ref, ind2, stamp, fnew, idx4, gc4, rows4, lbl4,
          semr, seml):
        cid = lax.axis_index("c")
        sid = lax.axis_index("s")
        w = sid * NC + cid
        lanes = lax.iota(jnp.int32, 16)
        pltpu.sync_copy(ind2.at[pl.ds(w * nch, nch)], idx4)
        sg = [pltpu.async_copy(stamp.at[idx4.at[c]], gc4.at[c], seml)
              for c in range(nch)]
        for d in sg:
            d.wait()
        rg = [pltpu.async_copy(fnew.at[gc4.at[c]], rows4.at[c], semr)
              for c in range(nch)]
        for c in range(nch):
            rg[c].wait()
            for j in range(_CH // 16):
                lv = plsc.load_gather(
                    rows4, [jnp.full((16,), c, jnp.int32),
                            j * 16 + lanes,
                            jnp.full((16,), 64, jnp.int32)])
                lbl4[c, pl.ds(j * 16, 16)] = plsc.bitcast(lv, jnp.int32)
        rs = [pltpu.async_copy(rows4.at[c], fb_ref.at[idx4.at[c]], semr)
              for c in range(nch)]
        ls = [pltpu.async_copy(lbl4.at[c], lb_ref.at[idx4.at[c]], seml)
              for c in range(nch)]
        for d in rs:
            d.wait()
        for d in ls:
            d.wait()

    return k


def _make_dense(C, F, FP, B, BB):
    G = B // BB

    def body(f_ref, fo_ref, c_ref, fn_ref, cs_ref):
        pid = pl.program_id(0)
        f = f_ref[...]
        foe = fo_ref[...]
        fo = foe[:, :F]
        ol = lax.bitcast_convert_type(foe[:, F:F + 1], jnp.int32)
        cen = c_ref[...]
        fn = f / (jnp.sqrt(jnp.sum(f * f, axis=1, keepdims=True)) + 1e-10)
        fnew = 0.5 * fo + 0.5 * fn
        fnew = fnew / (jnp.sqrt(jnp.sum(fnew * fnew, axis=1, keepdims=True))
                       + 1e-10)
        sims = lax.dot_general(fnew, cen, (((1,), (1,)), ((), ())),
                               preferred_element_type=jnp.float32)
        m = jnp.max(sims, axis=1, keepdims=True)
        cio = lax.broadcasted_iota(jnp.int32, sims.shape, 1)
        pick = jnp.where(sims == m, cio, jnp.int32(2 ** 30))
        lbl = jnp.min(pick, axis=1, keepdims=True)
        pad = jnp.zeros((BB, FP - F - 1), jnp.float32)
        fn_ref[...] = jnp.concatenate(
            [fnew, lax.bitcast_convert_type(lbl, jnp.float32), pad], axis=1)
        neq = (lbl != ol).astype(jnp.float32)
        s = jnp.sum(neq, axis=0, keepdims=True)

        @pl.when(pid == 0)
        def _():
            cs_ref[...] = jnp.zeros((1, 1), jnp.float32)

        cs_ref[...] += s * (1.0 / B)

    return pl.pallas_call(
        body,
        grid=(G,),
        in_specs=[
            pl.BlockSpec((BB, F), lambda i: (i, 0)),
            pl.BlockSpec((BB, FP), lambda i: (i, 0)),
            pl.BlockSpec((C, F), lambda i: (0, 0)),
        ],
        out_specs=[
            pl.BlockSpec((BB, FP), lambda i: (i, 0)),
            pl.BlockSpec((1, 1), lambda i: (0, 0)),
        ],
        out_shape=[
            jax.ShapeDtypeStruct((B, FP), jnp.float32),
            jax.ShapeDtypeStruct((1, 1), jnp.float32),
        ],
    )


def kernel(feature_bank, centroids, feature, label_bank, ind):
    L, F = feature_bank.shape
    C = centroids.shape[0]
    B = ind.shape[0]
    FP = 128
    NC, NS = _sc_info()
    BB = 1024

    fb128 = jnp.pad(feature_bank, ((0, 0), (0, FP - F)))

    ind2 = ind.astype(jnp.int32).reshape(B // _CH, _CH)
    stamp, lbc = _make_stamp(L, B, NC, NS)(label_bank, ind2)
    fo128 = _make_gather(L, FP, B, NC, NS)(fb128, label_bank, ind2)
    fnew128, cs = _make_dense(C, F, FP, B, BB)(feature, fo128, centroids)

    fb_ref = jax.new_ref(fb128)
    lb_ref = jax.new_ref(lbc)
    _make_scatter(L, FP, B, NC, NS)(fb_ref, lb_ref, ind2, stamp, fnew128)
    return fb_ref[...][:, :F], lb_ref[...], cs[0, 0]


# speedup vs baseline: 1.1826x; 1.0050x over previous
import functools

import jax
import jax.numpy as jnp
from jax import lax
from jax.experimental import pallas as pl
from jax.experimental.pallas import tpu as pltpu
from jax.experimental.pallas import tpu_sc as plsc

_CH = 128
_UNROLL = 8
_SC_PARAMS = dict(
    compiler_params=None,
)


def _sc_info():
    try:
        info = plsc.get_sparse_core_info()
        return info.num_cores, info.num_subcores
    except Exception:
        return 2, 16


def _sc_cp():
    return pltpu.CompilerParams(
        needs_layout_passes=False, use_tc_tiling_on_sc=False)


def _make_gather(L, FP, B, NC, NS):
    NW = NC * NS
    bpw = B // NW
    nch = bpw // _CH
    mesh = plsc.VectorSubcoreMesh(core_axis_name="c", subcore_axis_name="s")

    @functools.partial(
        pl.kernel,
        out_type=jax.ShapeDtypeStruct((B, FP), jnp.float32),
        mesh=mesh,
        scratch_types=[
            pltpu.VMEM((nch, _CH), jnp.int32),
            pltpu.VMEM((2, _CH, FP), jnp.float32),
            pltpu.VMEM((nch, _CH), jnp.int32),
            pltpu.SemaphoreType.DMA,
            pltpu.SemaphoreType.DMA,
        ],
        compiler_params=_sc_cp(),
    )
    def k(fb, lb, ind2, fo_out, idx4, rows2, lbl4, semr, seml):
        cid = lax.axis_index("c")
        sid = lax.axis_index("s")
        w = sid * NC + cid
        base = w * bpw
        lanes = lax.iota(jnp.int32, 16)
        pltpu.sync_copy(ind2.at[pl.ds(w * nch, nch)], idx4)
        lg = [pltpu.async_copy(lb.at[idx4.at[c]], lbl4.at[c], seml)
              for c in range(nch)]
        gd = [None, None]

        def fire(c):
            gd[c % 2] = pltpu.async_copy(fb.at[idx4.at[c]], rows2.at[c % 2],
                                         semr)

        def put(c):
            s = c % 2
            gd[s].wait()
            lg[c].wait()
            for j in range(_CH // 16):
                lv = plsc.bitcast(lbl4[c, pl.ds(j * 16, 16)], jnp.float32)
                plsc.store_scatter(
                    rows2, [jnp.full((16,), s, jnp.int32),
                            j * 16 + lanes,
                            jnp.full((16,), 64, jnp.int32)], lv)
            pltpu.sync_copy(rows2.at[s],
                            fo_out.at[pl.ds(base + c * _CH, _CH)])

        fire(0)
        for c in range(nch):
            if c + 1 < nch:
                fire(c + 1)
            put(c)

    return k


def _make_stamp(L, B, NC, NS):
    NW = NC * NS
    HB = B // 2
    lcw = ((L // NW) + 7) // 8 * 8
    mesh = plsc.VectorSubcoreMesh(core_axis_name="c", subcore_axis_name="s")

    @functools.partial(
        pl.kernel,
        out_type=(
            jax.ShapeDtypeStruct((L,), jnp.int32),
            jax.ShapeDtypeStruct((L,), jnp.int32),
        ),
        mesh=mesh,
        scratch_types=[
            pltpu.VMEM((HB // _CH, _CH), jnp.int32),
            pltpu.VMEM((L,), jnp.int32),
        ],
        compiler_params=_sc_cp(),
    )
    def k(lb, ind2, st_out, lbc_out, indh, stampv):
        cid = lax.axis_index("c")
        sid = lax.axis_index("s")
        w = sid * NC + cid
        lanes = lax.iota(jnp.int32, 16)
        lco = w * lcw
        ltail = L - (NW - 1) * lcw

        @pl.when(w < NW - 1)
        def _copy_body():
            pltpu.sync_copy(lb.at[pl.ds(lco, lcw)],
                            lbc_out.at[pl.ds(lco, lcw)])

        @pl.when(w == NW - 1)
        def _copy_tail():
            pltpu.sync_copy(lb.at[pl.ds(lco, ltail)],
                            lbc_out.at[pl.ds(lco, ltail)])

        @pl.when(w == 0)
        def _():
            nhr = HB // _CH
            for h in range(2):
                pltpu.sync_copy(ind2.at[pl.ds(h * nhr, nhr)], indh)

                def body(i, carry):
                    gbase = h * HB + i * _CH
                    idxs, bs = [], []
                    for u in range(_UNROLL):
                        idx16 = indh[i, pl.ds(u * 16, 16)]
                        b16 = gbase + u * 16 + lanes
                        plsc.store_scatter(stampv, [idx16], b16)
                        idxs.append(idx16)
                        bs.append(b16)
                    ok = None
                    for u in range(_UNROLL):
                        g = plsc.load_gather(stampv, [idxs[u]])
                        e = g == bs[u]
                        ok = e if ok is None else jnp.logical_and(ok, e)
                    dup = jnp.logical_not(jnp.all(ok))

                    @pl.when(dup)
                    def _fix():
                        for u in range(_UNROLL):
                            for kk in range(16):
                                plsc.store_scatter(stampv, [idxs[u]], bs[u],
                                                   mask=lanes == kk)

                    return carry

                lax.fori_loop(0, nhr, body, 0)
            pltpu.sync_copy(stampv, st_out)

    return k


def _make_scatter(L, FP, B, NC, NS):
    NW = NC * NS
    bpw = B // NW
    nch = bpw // _CH
    mesh = plsc.VectorSubcoreMesh(core_axis_name="c", subcore_axis_name="s")

    @functools.partial(
        pl.kernel,
        out_type=(),
        mesh=mesh,
        scratch_types=[
            pltpu.VMEM((nch, _CH), jnp.int32),
            pltpu.VMEM((nch, _CH), jnp.int32),
            pltpu.VMEM((nch, _CH, FP), jnp.float32),
            pltpu.VMEM((nch, _CH), jnp.int32),
            [pltpu.SemaphoreType.DMA] * 4,
            [pltpu.SemaphoreType.DMA] * 4,
            pltpu.SemaphoreType.DMA,
            pltpu.SemaphoreType.DMA,
        ],
        compiler_params=_sc_cp(),
    )
    def k(fb_ref, lb_ref, ind2, stamp, fnew, idx4, gc4, rows4, lbl4,
          sgs, rgs, semr, seml):
        cid = lax.axis_index("c")
        sid = lax.axis_index("s")
        w = sid * NC + cid
        lanes = lax.iota(jnp.int32, 16)
        pltpu.sync_copy(ind2.at[pl.ds(w * nch, nch)], idx4)
        sg = [pltpu.async_copy(stamp.at[idx4.at[c]], gc4.at[c], sgs[c])
              for c in range(nch)]
        rg = [None] * nch
        rs, ls = [], []
        for c in range(nch):
            sg[c].wait()
            rg[c] = pltpu.async_copy(fnew.at[gc4.at[c]], rows4.at[c], rgs[c])
        for c in range(nch):
            rg[c].wait()
            for j in range(_CH // 16):
                lv = plsc.load_gather(
                    rows4, [jnp.full((16,), c, jnp.int32),
                            j * 16 + lanes,
                            jnp.full((16,), 64, jnp.int32)])
                lbl4[c, pl.ds(j * 16, 16)] = plsc.bitcast(lv, jnp.int32)
            rs.append(pltpu.async_copy(rows4.at[c], fb_ref.at[idx4.at[c]],
                                       semr))
            ls.append(pltpu.async_copy(lbl4.at[c], lb_ref.at[idx4.at[c]],
                                       seml))
        for d in rs:
            d.wait()
        for d in ls:
            d.wait()

    return k


def _make_dense(C, F, FP, B, BB):
    G = B // BB

    def body(f_ref, fo_ref, c_ref, fn_ref, cs_ref):
        pid = pl.program_id(0)
        f = f_ref[...]
        foe = fo_ref[...]
        fo = foe[:, :F]
        ol = lax.bitcast_convert_type(foe[:, F:F + 1], jnp.int32)
        cen = c_ref[...]
        fn = f / (jnp.sqrt(jnp.sum(f * f, axis=1, keepdims=True)) + 1e-10)
        fnew = 0.5 * fo + 0.5 * fn
        fnew = fnew / (jnp.sqrt(jnp.sum(fnew * fnew, axis=1, keepdims=True))
                       + 1e-10)
        sims = lax.dot_general(fnew, cen, (((1,), (1,)), ((), ())),
                               preferred_element_type=jnp.float32)
        m = jnp.max(sims, axis=1, keepdims=True)
        cio = lax.broadcasted_iota(jnp.int32, sims.shape, 1)
        pick = jnp.where(sims == m, cio, jnp.int32(2 ** 30))
        lbl = jnp.min(pick, axis=1, keepdims=True)
        pad = jnp.zeros((BB, FP - F - 1), jnp.float32)
        fn_ref[...] = jnp.concatenate(
            [fnew, lax.bitcast_convert_type(lbl, jnp.float32), pad], axis=1)
        neq = (lbl != ol).astype(jnp.float32)
        s = jnp.sum(neq, axis=0, keepdims=True)

        @pl.when(pid == 0)
        def _():
            cs_ref[...] = jnp.zeros((1, 1), jnp.float32)

        cs_ref[...] += s * (1.0 / B)

    return pl.pallas_call(
        body,
        grid=(G,),
        in_specs=[
            pl.BlockSpec((BB, F), lambda i: (i, 0)),
            pl.BlockSpec((BB, FP), lambda i: (i, 0)),
            pl.BlockSpec((C, F), lambda i: (0, 0)),
        ],
        out_specs=[
            pl.BlockSpec((BB, FP), lambda i: (i, 0)),
            pl.BlockSpec((1, 1), lambda i: (0, 0)),
        ],
        out_shape=[
            jax.ShapeDtypeStruct((B, FP), jnp.float32),
            jax.ShapeDtypeStruct((1, 1), jnp.float32),
        ],
    )


def kernel(feature_bank, centroids, feature, label_bank, ind):
    L, F = feature_bank.shape
    C = centroids.shape[0]
    B = ind.shape[0]
    FP = 128
    NC, NS = _sc_info()
    BB = 1024

    fb128 = jnp.pad(feature_bank, ((0, 0), (0, FP - F)))

    ind2 = ind.astype(jnp.int32).reshape(B // _CH, _CH)
    stamp, lbc = _make_stamp(L, B, NC, NS)(label_bank, ind2)
    fo128 = _make_gather(L, FP, B, NC, NS)(fb128, label_bank, ind2)
    fnew128, cs = _make_dense(C, F, FP, B, BB)(feature, fo128, centroids)

    fb_ref = jax.new_ref(fb128)
    lb_ref = jax.new_ref(lbc)
    _make_scatter(L, FP, B, NC, NS)(fb_ref, lb_ref, ind2, stamp, fnew128)
    return fb_ref[...][:, :F], lb_ref[...], cs[0, 0]


# speedup vs baseline: 1.1905x; 1.0067x over previous
import functools

import jax
import jax.numpy as jnp
from jax import lax
from jax.experimental import pallas as pl
from jax.experimental.pallas import tpu as pltpu
from jax.experimental.pallas import tpu_sc as plsc

_CH = 128
_UNROLL = 8
_SC_PARAMS = dict(
    compiler_params=None,
)


def _sc_info():
    try:
        info = plsc.get_sparse_core_info()
        return info.num_cores, info.num_subcores
    except Exception:
        return 2, 16


def _sc_cp():
    return pltpu.CompilerParams(
        needs_layout_passes=False, use_tc_tiling_on_sc=False)


def _make_gather(L, FP, B, NC, NS):
    NW = NC * NS
    bpw = B // NW
    nch = bpw // _CH
    mesh = plsc.VectorSubcoreMesh(core_axis_name="c", subcore_axis_name="s")

    @functools.partial(
        pl.kernel,
        out_type=jax.ShapeDtypeStruct((B, FP), jnp.float32),
        mesh=mesh,
        scratch_types=[
            pltpu.VMEM((nch, _CH), jnp.int32),
            pltpu.VMEM((2, _CH, FP), jnp.float32),
            pltpu.VMEM((nch, _CH), jnp.int32),
            pltpu.SemaphoreType.DMA,
            pltpu.SemaphoreType.DMA,
        ],
        compiler_params=_sc_cp(),
    )
    def k(fb, lb, ind2, fo_out, idx4, rows2, lbl4, semr, seml):
        cid = lax.axis_index("c")
        sid = lax.axis_index("s")
        w = sid * NC + cid
        base = w * bpw
        lanes = lax.iota(jnp.int32, 16)
        pltpu.sync_copy(ind2.at[pl.ds(w * nch, nch)], idx4)
        lg = [pltpu.async_copy(lb.at[idx4.at[c]], lbl4.at[c], seml)
              for c in range(nch)]
        gd = [None, None]

        def fire(c):
            gd[c % 2] = pltpu.async_copy(fb.at[idx4.at[c]], rows2.at[c % 2],
                                         semr)

        def put(c):
            s = c % 2
            gd[s].wait()
            lg[c].wait()
            for j in range(_CH // 16):
                lv = plsc.bitcast(lbl4[c, pl.ds(j * 16, 16)], jnp.float32)
                plsc.store_scatter(
                    rows2, [jnp.full((16,), s, jnp.int32),
                            j * 16 + lanes,
                            jnp.full((16,), 64, jnp.int32)], lv)
            pltpu.sync_copy(rows2.at[s],
                            fo_out.at[pl.ds(base + c * _CH, _CH)])

        fire(0)
        for c in range(nch):
            if c + 1 < nch:
                fire(c + 1)
            put(c)

    return k


def _make_stamp(L, B, NC, NS):
    NW = NC * NS
    HB = B // 2
    lcw = ((L // NW) + 7) // 8 * 8
    mesh = plsc.VectorSubcoreMesh(core_axis_name="c", subcore_axis_name="s")

    @functools.partial(
        pl.kernel,
        out_type=(
            jax.ShapeDtypeStruct((L,), jnp.int32),
            jax.ShapeDtypeStruct((L,), jnp.int32),
        ),
        mesh=mesh,
        scratch_types=[
            pltpu.VMEM((HB // _CH, _CH), jnp.int32),
            pltpu.VMEM((L,), jnp.int32),
        ],
        compiler_params=_sc_cp(),
    )
    def k(lb, ind2, st_out, lbc_out, indh, stampv):
        cid = lax.axis_index("c")
        sid = lax.axis_index("s")
        w = sid * NC + cid
        lanes = lax.iota(jnp.int32, 16)
        lco = w * lcw
        ltail = L - (NW - 1) * lcw

        @pl.when(w < NW - 1)
        def _copy_body():
            pltpu.sync_copy(lb.at[pl.ds(lco, lcw)],
                            lbc_out.at[pl.ds(lco, lcw)])

        @pl.when(w == NW - 1)
        def _copy_tail():
            pltpu.sync_copy(lb.at[pl.ds(lco, ltail)],
                            lbc_out.at[pl.ds(lco, ltail)])

        @pl.when(w == 0)
        def _():
            nhr = HB // _CH
            for h in range(2):
                pltpu.sync_copy(ind2.at[pl.ds(h * nhr, nhr)], indh)

                def body(i, carry):
                    gbase = h * HB + i * _CH
                    idxs, bs = [], []
                    for u in range(_UNROLL):
                        idx16 = indh[i, pl.ds(u * 16, 16)]
                        b16 = gbase + u * 16 + lanes
                        plsc.store_scatter(stampv, [idx16], b16)
                        idxs.append(idx16)
                        bs.append(b16)
                    ok = None
                    for u in range(_UNROLL):
                        g = plsc.load_gather(stampv, [idxs[u]])
                        e = g == bs[u]
                        ok = e if ok is None else jnp.logical_and(ok, e)
                    dup = jnp.logical_not(jnp.all(ok))

                    @pl.when(dup)
                    def _fix():
                        for u in range(_UNROLL):
                            for kk in range(16):
                                plsc.store_scatter(stampv, [idxs[u]], bs[u],
                                                   mask=lanes == kk)

                    return carry

                lax.fori_loop(0, nhr, body, 0)
            pltpu.sync_copy(stampv, st_out)

    return k


def _make_scatter(L, FP, B, NC, NS):
    NW = NC * NS
    bpw = B // NW
    nch = bpw // _CH
    mesh = plsc.VectorSubcoreMesh(core_axis_name="c", subcore_axis_name="s")

    @functools.partial(
        pl.kernel,
        out_type=(),
        mesh=mesh,
        scratch_types=[
            pltpu.VMEM((nch, _CH), jnp.int32),
            pltpu.VMEM((nch, _CH), jnp.int32),
            pltpu.VMEM((nch, _CH, FP), jnp.float32),
            pltpu.VMEM((nch, _CH), jnp.int32),
            [pltpu.SemaphoreType.DMA] * 4,
            [pltpu.SemaphoreType.DMA] * 4,
            pltpu.SemaphoreType.DMA,
            pltpu.SemaphoreType.DMA,
        ],
        compiler_params=_sc_cp(),
    )
    def k(fb_ref, lb_ref, ind2, stamp, fnew, idx4, gc4, rows4, lbl4,
          sgs, rgs, semr, seml):
        cid = lax.axis_index("c")
        sid = lax.axis_index("s")
        w = sid * NC + cid
        lanes = lax.iota(jnp.int32, 16)
        pltpu.sync_copy(ind2.at[pl.ds(w * nch, nch)], idx4)
        sg = [pltpu.async_copy(stamp.at[idx4.at[c]], gc4.at[c], sgs[c])
              for c in range(nch)]
        rg = [None] * nch
        rs, ls = [], []
        for c in range(nch):
            sg[c].wait()
            rg[c] = pltpu.async_copy(fnew.at[gc4.at[c]], rows4.at[c], rgs[c])
        for c in range(nch):
            rg[c].wait()
            for j in range(_CH // 16):
                lv = plsc.load_gather(
                    rows4, [jnp.full((16,), c, jnp.int32),
                            j * 16 + lanes,
                            jnp.full((16,), 64, jnp.int32)])
                lbl4[c, pl.ds(j * 16, 16)] = plsc.bitcast(lv, jnp.int32)
            rs.append(pltpu.async_copy(rows4.at[c], fb_ref.at[idx4.at[c]],
                                       semr))
            ls.append(pltpu.async_copy(lbl4.at[c], lb_ref.at[idx4.at[c]],
                                       seml))
        for d in rs:
            d.wait()
        for d in ls:
            d.wait()

    return k


def _make_dense(C, F, FP, B, BB):
    G = B // BB

    def body(f_ref, fo_ref, c_ref, fn_ref, cs_ref):
        pid = pl.program_id(0)
        f = f_ref[...]
        foe = fo_ref[...]
        fo = foe[:, :F]
        ol = lax.bitcast_convert_type(foe[:, F:F + 1], jnp.int32)
        cen = c_ref[...]
        fn = f / (jnp.sqrt(jnp.sum(f * f, axis=1, keepdims=True)) + 1e-10)
        fnew = 0.5 * fo + 0.5 * fn
        fnew = fnew / (jnp.sqrt(jnp.sum(fnew * fnew, axis=1, keepdims=True))
                       + 1e-10)
        sims = lax.dot_general(fnew, cen, (((1,), (1,)), ((), ())),
                               preferred_element_type=jnp.float32)
        lbl = jnp.argmax(sims, axis=1).astype(jnp.int32)[:, None]
        pad = jnp.zeros((BB, FP - F - 1), jnp.float32)
        fn_ref[...] = jnp.concatenate(
            [fnew, lax.bitcast_convert_type(lbl, jnp.float32), pad], axis=1)
        neq = (lbl != ol).astype(jnp.float32)
        s = jnp.sum(neq, axis=0, keepdims=True)

        @pl.when(pid == 0)
        def _():
            cs_ref[...] = jnp.zeros((1, 1), jnp.float32)

        cs_ref[...] += s * (1.0 / B)

    return pl.pallas_call(
        body,
        grid=(G,),
        in_specs=[
            pl.BlockSpec((BB, F), lambda i: (i, 0)),
            pl.BlockSpec((BB, FP), lambda i: (i, 0)),
            pl.BlockSpec((C, F), lambda i: (0, 0)),
        ],
        out_specs=[
            pl.BlockSpec((BB, FP), lambda i: (i, 0)),
            pl.BlockSpec((1, 1), lambda i: (0, 0)),
        ],
        out_shape=[
            jax.ShapeDtypeStruct((B, FP), jnp.float32),
            jax.ShapeDtypeStruct((1, 1), jnp.float32),
        ],
    )


def kernel(feature_bank, centroids, feature, label_bank, ind):
    L, F = feature_bank.shape
    C = centroids.shape[0]
    B = ind.shape[0]
    FP = 128
    NC, NS = _sc_info()
    BB = 1024

    fb128 = jnp.pad(feature_bank, ((0, 0), (0, FP - F)))

    ind2 = ind.astype(jnp.int32).reshape(B // _CH, _CH)
    stamp, lbc = _make_stamp(L, B, NC, NS)(label_bank, ind2)
    fo128 = _make_gather(L, FP, B, NC, NS)(fb128, label_bank, ind2)
    fnew128, cs = _make_dense(C, F, FP, B, BB)(feature, fo128, centroids)

    fb_ref = jax.new_ref(fb128)
    lb_ref = jax.new_ref(lbc)
    _make_scatter(L, FP, B, NC, NS)(fb_ref, lb_ref, ind2, stamp, fnew128)
    return fb_ref[...][:, :F], lb_ref[...], cs[0, 0]


# speedup vs baseline: 1.1911x; 1.0005x over previous
import functools

import jax
import jax.numpy as jnp
from jax import lax
from jax.experimental import pallas as pl
from jax.experimental.pallas import tpu as pltpu
from jax.experimental.pallas import tpu_sc as plsc

_CH = 128
_UNROLL = 8
_SC_PARAMS = dict(
    compiler_params=None,
)


def _sc_info():
    try:
        info = plsc.get_sparse_core_info()
        return info.num_cores, info.num_subcores
    except Exception:
        return 2, 16


def _sc_cp():
    return pltpu.CompilerParams(
        needs_layout_passes=False, use_tc_tiling_on_sc=False)


def _make_gather(L, FP, B, NC, NS):
    NW = NC * NS
    bpw = B // NW
    nch = bpw // _CH
    mesh = plsc.VectorSubcoreMesh(core_axis_name="c", subcore_axis_name="s")

    @functools.partial(
        pl.kernel,
        out_type=jax.ShapeDtypeStruct((B, FP), jnp.float32),
        mesh=mesh,
        scratch_types=[
            pltpu.VMEM((nch, _CH), jnp.int32),
            pltpu.VMEM((2, _CH, FP), jnp.float32),
            pltpu.VMEM((nch, _CH), jnp.int32),
            pltpu.SemaphoreType.DMA,
            pltpu.SemaphoreType.DMA,
        ],
        compiler_params=_sc_cp(),
    )
    def k(fb, lb, ind2, fo_out, idx4, rows2, lbl4, semr, seml):
        cid = lax.axis_index("c")
        sid = lax.axis_index("s")
        w = sid * NC + cid
        base = w * bpw
        lanes = lax.iota(jnp.int32, 16)
        pltpu.sync_copy(ind2.at[pl.ds(w * nch, nch)], idx4)
        lg = [pltpu.async_copy(lb.at[idx4.at[c]], lbl4.at[c], seml)
              for c in range(nch)]
        gd = [None, None]

        def fire(c):
            gd[c % 2] = pltpu.async_copy(fb.at[idx4.at[c]], rows2.at[c % 2],
                                         semr)

        def put(c):
            s = c % 2
            gd[s].wait()
            lg[c].wait()
            for j in range(_CH // 16):
                lv = plsc.bitcast(lbl4[c, pl.ds(j * 16, 16)], jnp.float32)
                plsc.store_scatter(
                    rows2, [jnp.full((16,), s, jnp.int32),
                            j * 16 + lanes,
                            jnp.full((16,), 64, jnp.int32)], lv)
            pltpu.sync_copy(rows2.at[s],
                            fo_out.at[pl.ds(base + c * _CH, _CH)])

        fire(0)
        for c in range(nch):
            if c + 1 < nch:
                fire(c + 1)
            put(c)

    return k


def _make_stamp(L, B, NC, NS):
    NW = NC * NS
    HB = B // 2
    lcw = ((L // NW) + 7) // 8 * 8
    mesh = plsc.VectorSubcoreMesh(core_axis_name="c", subcore_axis_name="s")

    @functools.partial(
        pl.kernel,
        out_type=(
            jax.ShapeDtypeStruct((L,), jnp.int32),
            jax.ShapeDtypeStruct((L,), jnp.int32),
        ),
        mesh=mesh,
        scratch_types=[
            pltpu.VMEM((HB // _CH, _CH), jnp.int32),
            pltpu.VMEM((L,), jnp.int32),
        ],
        compiler_params=_sc_cp(),
    )
    def k(lb, ind2, st_out, lbc_out, indh, stampv):
        cid = lax.axis_index("c")
        sid = lax.axis_index("s")
        w = sid * NC + cid
        lanes = lax.iota(jnp.int32, 16)
        lco = w * lcw
        ltail = L - (NW - 1) * lcw

        @pl.when(w < NW - 1)
        def _copy_body():
            pltpu.sync_copy(lb.at[pl.ds(lco, lcw)],
                            lbc_out.at[pl.ds(lco, lcw)])

        @pl.when(w == NW - 1)
        def _copy_tail():
            pltpu.sync_copy(lb.at[pl.ds(lco, ltail)],
                            lbc_out.at[pl.ds(lco, ltail)])

        @pl.when(w == 0)
        def _():
            nhr = HB // _CH
            for h in range(2):
                pltpu.sync_copy(ind2.at[pl.ds(h * nhr, nhr)], indh)

                def body(i, carry):
                    gbase = h * HB + i * _CH
                    idxs, bs = [], []
                    for u in range(_UNROLL):
                        idx16 = indh[i, pl.ds(u * 16, 16)]
                        b16 = gbase + u * 16 + lanes
                        plsc.store_scatter(stampv, [idx16], b16)
                        idxs.append(idx16)
                        bs.append(b16)
                    ok = None
                    for u in range(_UNROLL):
                        g = plsc.load_gather(stampv, [idxs[u]])
                        e = g == bs[u]
                        ok = e if ok is None else jnp.logical_and(ok, e)
                    dup = jnp.logical_not(jnp.all(ok))

                    @pl.when(dup)
                    def _fix():
                        for u in range(_UNROLL):
                            for kk in range(16):
                                plsc.store_scatter(stampv, [idxs[u]], bs[u],
                                                   mask=lanes == kk)

                    return carry

                lax.fori_loop(0, nhr, body, 0)
            pltpu.sync_copy(stampv, st_out)

    return k


def _make_scatter(L, FP, B, NC, NS):
    NW = NC * NS
    bpw = B // NW
    nch = bpw // _CH
    mesh = plsc.VectorSubcoreMesh(core_axis_name="c", subcore_axis_name="s")

    @functools.partial(
        pl.kernel,
        out_type=(),
        mesh=mesh,
        scratch_types=[
            pltpu.VMEM((nch, _CH), jnp.int32),
            pltpu.VMEM((nch, _CH), jnp.int32),
            pltpu.VMEM((nch, _CH, FP), jnp.float32),
            pltpu.VMEM((nch, _CH), jnp.int32),
            [pltpu.SemaphoreType.DMA] * 4,
            [pltpu.SemaphoreType.DMA] * 4,
            pltpu.SemaphoreType.DMA,
            pltpu.SemaphoreType.DMA,
        ],
        compiler_params=_sc_cp(),
    )
    def k(fb_ref, lb_ref, ind2, stamp, fnew, idx4, gc4, rows4, lbl4,
          sgs, rgs, semr, seml):
        cid = lax.axis_index("c")
        sid = lax.axis_index("s")
        w = sid * NC + cid
        lanes = lax.iota(jnp.int32, 16)
        pltpu.sync_copy(ind2.at[pl.ds(w * nch, nch)], idx4)
        sg = [pltpu.async_copy(stamp.at[idx4.at[c]], gc4.at[c], sgs[c])
              for c in range(nch)]
        rg = [None] * nch
        rs, ls = [], []
        for c in range(nch):
            sg[c].wait()
            rg[c] = pltpu.async_copy(fnew.at[gc4.at[c]], rows4.at[c], rgs[c])
        for c in range(nch):
            rg[c].wait()
            for j in range(_CH // 16):
                lv = plsc.load_gather(
                    rows4, [jnp.full((16,), c, jnp.int32),
                            j * 16 + lanes,
                            jnp.full((16,), 64, jnp.int32)])
                lbl4[c, pl.ds(j * 16, 16)] = plsc.bitcast(lv, jnp.int32)
            rs.append(pltpu.async_copy(rows4.at[c], fb_ref.at[idx4.at[c]],
                                       semr))
            ls.append(pltpu.async_copy(lbl4.at[c], lb_ref.at[idx4.at[c]],
                                       seml))
        for d in rs:
            d.wait()
        for d in ls:
            d.wait()

    return k


def _make_dense(C, F, FP, B, BB):
    G = B // BB

    def body(f_ref, fo_ref, c_ref, fn_ref, cs_ref):
        pid = pl.program_id(0)
        f = f_ref[...]
        foe = fo_ref[...]
        fo = foe[:, :F]
        ol = lax.bitcast_convert_type(foe[:, F:F + 1], jnp.int32)
        cen = c_ref[...]
        fn = f / (jnp.sqrt(jnp.sum(f * f, axis=1, keepdims=True)) + 1e-10)
        fnew = 0.5 * fo + 0.5 * fn
        fnew = fnew / (jnp.sqrt(jnp.sum(fnew * fnew, axis=1, keepdims=True))
                       + 1e-10)
        sims = lax.dot_general(fnew, cen, (((1,), (1,)), ((), ())),
                               preferred_element_type=jnp.float32)
        lbl = jnp.argmax(sims, axis=1).astype(jnp.int32)[:, None]
        pad = jnp.zeros((BB, FP - F - 1), jnp.float32)
        fn_ref[...] = jnp.concatenate(
            [fnew, lax.bitcast_convert_type(lbl, jnp.float32), pad], axis=1)
        neq = (lbl != ol).astype(jnp.float32)
        s = jnp.sum(neq, axis=0, keepdims=True)

        @pl.when(pid == 0)
        def _():
            cs_ref[...] = jnp.zeros((1, 1), jnp.float32)

        cs_ref[...] += s * (1.0 / B)

    return pl.pallas_call(
        body,
        grid=(G,),
        in_specs=[
            pl.BlockSpec((BB, F), lambda i: (i, 0)),
            pl.BlockSpec((BB, FP), lambda i: (i, 0)),
            pl.BlockSpec((C, F), lambda i: (0, 0)),
        ],
        out_specs=[
            pl.BlockSpec((BB, FP), lambda i: (i, 0)),
            pl.BlockSpec((1, 1), lambda i: (0, 0)),
        ],
        out_shape=[
            jax.ShapeDtypeStruct((B, FP), jnp.float32),
            jax.ShapeDtypeStruct((1, 1), jnp.float32),
        ],
    )


def kernel(feature_bank, centroids, feature, label_bank, ind):
    L, F = feature_bank.shape
    C = centroids.shape[0]
    B = ind.shape[0]
    FP = 128
    NC, NS = _sc_info()
    BB = 2048

    fb128 = jnp.pad(feature_bank, ((0, 0), (0, FP - F)))

    ind2 = ind.astype(jnp.int32).reshape(B // _CH, _CH)
    stamp, lbc = _make_stamp(L, B, NC, NS)(label_bank, ind2)
    fo128 = _make_gather(L, FP, B, NC, NS)(fb128, label_bank, ind2)
    fnew128, cs = _make_dense(C, F, FP, B, BB)(feature, fo128, centroids)

    fb_ref = jax.new_ref(fb128)
    lb_ref = jax.new_ref(lbc)
    _make_scatter(L, FP, B, NC, NS)(fb_ref, lb_ref, ind2, stamp, fnew128)
    return fb_ref[...][:, :F], lb_ref[...], cs[0, 0]


# speedup vs baseline: 1.1950x; 1.0033x over previous
import functools

import jax
import jax.numpy as jnp
from jax import lax
from jax.experimental import pallas as pl
from jax.experimental.pallas import tpu as pltpu
from jax.experimental.pallas import tpu_sc as plsc

_CH = 128
_UNROLL = 8


def _sc_info():
    try:
        info = plsc.get_sparse_core_info()
        return info.num_cores, info.num_subcores
    except Exception:
        return 2, 16


def _sc_cp():
    return pltpu.CompilerParams(
        needs_layout_passes=False, use_tc_tiling_on_sc=False)


def _make_gather(L, FP, B, NC, NS):
    NW = NC * NS
    bpw = B // NW
    nch = bpw // _CH
    mesh = plsc.VectorSubcoreMesh(core_axis_name="c", subcore_axis_name="s")

    @functools.partial(
        pl.kernel,
        out_type=jax.ShapeDtypeStruct((B, FP), jnp.float32),
        mesh=mesh,
        scratch_types=[
            pltpu.VMEM((nch, _CH), jnp.int32),
            pltpu.VMEM((2, _CH, FP), jnp.float32),
            pltpu.VMEM((nch, _CH), jnp.int32),
            pltpu.SemaphoreType.DMA,
            pltpu.SemaphoreType.DMA,
        ],
        compiler_params=_sc_cp(),
    )
    def k(fb, lb, ind2, fo_out, idx4, rows2, lbl4, semr, seml):
        cid = lax.axis_index("c")
        sid = lax.axis_index("s")
        w = sid * NC + cid
        base = w * bpw
        lanes = lax.iota(jnp.int32, 16)
        pltpu.sync_copy(ind2.at[pl.ds(w * nch, nch)], idx4)
        lg = [pltpu.async_copy(lb.at[idx4.at[c]], lbl4.at[c], seml)
              for c in range(nch)]
        gd = [None, None]

        def fire(c):
            gd[c % 2] = pltpu.async_copy(fb.at[idx4.at[c]], rows2.at[c % 2],
                                         semr)

        def put(c):
            s = c % 2
            gd[s].wait()
            lg[c].wait()
            for j in range(_CH // 16):
                lv = plsc.bitcast(lbl4[c, pl.ds(j * 16, 16)], jnp.float32)
                plsc.store_scatter(
                    rows2, [jnp.full((16,), s, jnp.int32),
                            j * 16 + lanes,
                            jnp.full((16,), 64, jnp.int32)], lv)
            pltpu.sync_copy(rows2.at[s],
                            fo_out.at[pl.ds(base + c * _CH, _CH)])

        fire(0)
        for c in range(nch):
            if c + 1 < nch:
                fire(c + 1)
            put(c)

    return k


def _make_stamp(L, B, NC, NS):
    NW = NC * NS
    HB = B // 2
    lcw = ((L // NW) + 7) // 8 * 8
    mesh = plsc.VectorSubcoreMesh(core_axis_name="c", subcore_axis_name="s")

    @functools.partial(
        pl.kernel,
        out_type=(
            jax.ShapeDtypeStruct((L,), jnp.int32),
            jax.ShapeDtypeStruct((L,), jnp.int32),
        ),
        mesh=mesh,
        scratch_types=[
            pltpu.VMEM((HB // _CH, _CH), jnp.int32),
            pltpu.VMEM((L,), jnp.int32),
        ],
        compiler_params=_sc_cp(),
    )
    def k(lb, ind2, st_out, lbc_out, indh, stampv):
        cid = lax.axis_index("c")
        sid = lax.axis_index("s")
        w = sid * NC + cid
        lanes = lax.iota(jnp.int32, 16)
        lco = w * lcw
        ltail = L - (NW - 1) * lcw

        @pl.when(w < NW - 1)
        def _copy_body():
            pltpu.sync_copy(lb.at[pl.ds(lco, lcw)],
                            lbc_out.at[pl.ds(lco, lcw)])

        @pl.when(w == NW - 1)
        def _copy_tail():
            pltpu.sync_copy(lb.at[pl.ds(lco, ltail)],
                            lbc_out.at[pl.ds(lco, ltail)])

        @pl.when(w == 0)
        def _():
            nhr = HB // _CH
            for h in range(2):
                pltpu.sync_copy(ind2.at[pl.ds(h * nhr, nhr)], indh)

                def body(i, carry):
                    gbase = h * HB + i * _CH
                    idxs, bs = [], []
                    for u in range(_UNROLL):
                        idx16 = indh[i, pl.ds(u * 16, 16)]
                        b16 = gbase + u * 16 + lanes
                        plsc.store_scatter(stampv, [idx16], b16)
                        idxs.append(idx16)
                        bs.append(b16)
                    ok = None
                    for u in range(_UNROLL):
                        g = plsc.load_gather(stampv, [idxs[u]])
                        e = g == bs[u]
                        ok = e if ok is None else jnp.logical_and(ok, e)
                    dup = jnp.logical_not(jnp.all(ok))

                    @pl.when(dup)
                    def _fix():
                        for u in range(_UNROLL):
                            for kk in range(16):
                                plsc.store_scatter(stampv, [idxs[u]], bs[u],
                                                   mask=lanes == kk)

                    return carry

                lax.fori_loop(0, nhr, body, 0)
            pltpu.sync_copy(stampv, st_out)

    return k


def _make_scatter(L, FP, B, NC, NS):
    NW = NC * NS
    bpw = B // NW
    nch = bpw // _CH
    mesh = plsc.VectorSubcoreMesh(core_axis_name="c", subcore_axis_name="s")

    @functools.partial(
        pl.kernel,
        out_type=(),
        mesh=mesh,
        scratch_types=[
            pltpu.VMEM((nch, _CH), jnp.int32),
            pltpu.VMEM((nch, _CH), jnp.int32),
            pltpu.VMEM((nch, _CH, FP), jnp.float32),
            pltpu.VMEM((nch, _CH), jnp.int32),
            [pltpu.SemaphoreType.DMA] * 4,
            [pltpu.SemaphoreType.DMA] * 4,
            pltpu.SemaphoreType.DMA,
            pltpu.SemaphoreType.DMA,
        ],
        compiler_params=_sc_cp(),
    )
    def k(fb_ref, lb_ref, ind2, stamp, fnew, idx4, gc4, rows4, lbl4,
          sgs, rgs, semr, seml):
        cid = lax.axis_index("c")
        sid = lax.axis_index("s")
        w = sid * NC + cid
        lanes = lax.iota(jnp.int32, 16)
        pltpu.sync_copy(ind2.at[pl.ds(w * nch, nch)], idx4)
        sg = [pltpu.async_copy(stamp.at[idx4.at[c]], gc4.at[c], sgs[c])
              for c in range(nch)]
        rg = [None] * nch
        rs, ls = [], []
        for c in range(nch):
            sg[c].wait()
            rg[c] = pltpu.async_copy(fnew.at[gc4.at[c]], rows4.at[c], rgs[c])
        for c in range(nch):
            rg[c].wait()
            for j in range(_CH // 16):
                lv = plsc.load_gather(
                    rows4, [jnp.full((16,), c, jnp.int32),
                            j * 16 + lanes,
                            jnp.full((16,), 64, jnp.int32)])
                lbl4[c, pl.ds(j * 16, 16)] = plsc.bitcast(lv, jnp.int32)
            rs.append(pltpu.async_copy(rows4.at[c], fb_ref.at[idx4.at[c]],
                                       semr))
            ls.append(pltpu.async_copy(lbl4.at[c], lb_ref.at[idx4.at[c]],
                                       seml))
        for d in rs:
            d.wait()
        for d in ls:
            d.wait()

    return k


def _make_dense(C, F, FP, B, BB):
    G = B // BB

    def body(f_ref, fo_ref, c_ref, fn_ref, cs_ref):
        pid = pl.program_id(0)
        f = f_ref[...]
        foe = fo_ref[...]
        fo = foe[:, :F]
        ol = lax.bitcast_convert_type(foe[:, F:F + 1], jnp.int32)
        cen = c_ref[...]
        fn = f / (jnp.sqrt(jnp.sum(f * f, axis=1, keepdims=True)) + 1e-10)
        fnew = 0.5 * fo + 0.5 * fn
        fnew = fnew / (jnp.sqrt(jnp.sum(fnew * fnew, axis=1, keepdims=True))
                       + 1e-10)
        sims = lax.dot_general(fnew, cen, (((1,), (1,)), ((), ())),
                               preferred_element_type=jnp.float32)
        lbl = jnp.argmax(sims, axis=1).astype(jnp.int32)[:, None]
        pad = jnp.zeros((BB, FP - F - 1), jnp.float32)
        fn_ref[...] = jnp.concatenate(
            [fnew, lax.bitcast_convert_type(lbl, jnp.float32), pad], axis=1)
        neq = (lbl != ol).astype(jnp.float32)
        s = jnp.sum(neq, axis=0, keepdims=True)

        @pl.when(pid == 0)
        def _():
            cs_ref[...] = jnp.zeros((1, 1), jnp.float32)

        cs_ref[...] += s * (1.0 / B)

    return pl.pallas_call(
        body,
        grid=(G,),
        in_specs=[
            pl.BlockSpec((BB, F), lambda i: (i, 0)),
            pl.BlockSpec((BB, FP), lambda i: (i, 0)),
            pl.BlockSpec((C, F), lambda i: (0, 0)),
        ],
        out_specs=[
            pl.BlockSpec((BB, FP), lambda i: (i, 0)),
            pl.BlockSpec((1, 1), lambda i: (0, 0)),
        ],
        out_shape=[
            jax.ShapeDtypeStruct((B, FP), jnp.float32),
            jax.ShapeDtypeStruct((1, 1), jnp.float32),
        ],
    )


def kernel(feature_bank, centroids, feature, label_bank, ind):
    L, F = feature_bank.shape
    C = centroids.shape[0]
    B = ind.shape[0]
    FP = 128
    NC, NS = _sc_info()
    BB = 2048

    fb128 = jnp.pad(feature_bank, ((0, 0), (0, FP - F)))

    ind2 = ind.astype(jnp.int32).reshape(B // _CH, _CH)
    stamp, lbc = _make_stamp(L, B, NC, NS)(label_bank, ind2)
    fo128 = _make_gather(L, FP, B, NC, NS)(fb128, label_bank, ind2)
    fnew128, cs = _make_dense(C, F, FP, B, BB)(feature, fo128, centroids)

    fb_ref = jax.new_ref(fb128)
    lb_ref = jax.new_ref(lbc)
    _make_scatter(L, FP, B, NC, NS)(fb_ref, lb_ref, ind2, stamp, fnew128)
    return fb_ref[...][:, :F], lb_ref[...], cs[0, 0]
